# baseline (speedup 1.0000x reference)
import functools
import math

import jax
import jax.numpy as jnp
from jax import lax
from jax.experimental import pallas as pl
from jax.experimental.pallas import tpu as pltpu
from jax.experimental.pallas import tpu_sc as plsc

D_EMBED = 64
SCALE = math.sqrt(D_EMBED)
NUM_CORES = 2
NUM_SUBCORES = 16
NUM_WORKERS = NUM_CORES * NUM_SUBCORES
LANES = 16
NBUF = 2
ROW_UNROLL = 4


def _build_sc_gather(xb: int, xt: int):
    assert xb % (NUM_WORKERS * NBUF) == 0
    rows_per_worker = xb // NUM_WORKERS

    mesh = plsc.VectorSubcoreMesh(core_axis_name="c", subcore_axis_name="s")

    @functools.partial(
        pl.kernel,
        out_type=jax.ShapeDtypeStruct((xb, xt, D_EMBED), jnp.float32),
        mesh=mesh,
        scratch_types=[
            pltpu.VMEM((rows_per_worker, xt), jnp.int32),
            pltpu.VMEM((NBUF, xt, D_EMBED), jnp.float32),
            pltpu.SemaphoreType.DMA((NBUF,)),
            pltpu.SemaphoreType.DMA((NBUF,)),
        ],
        compiler_params=pltpu.CompilerParams(use_tc_tiling_on_sc=False),
    )
    def sc_gather(x_hbm, tab_hbm, out_hbm, idx_v, rows_v, gsem, ssem):
        wid = lax.axis_index("s") * NUM_CORES + lax.axis_index("c")
        base = wid * rows_per_worker
        pltpu.sync_copy(x_hbm.at[pl.ds(base, rows_per_worker)], idx_v)

        def gather(r, b):
            return pltpu.make_async_copy(
                tab_hbm.at[idx_v.at[r]],
                rows_v.at[b],
                gsem.at[b],
            )

        def store(r, b):
            return pltpu.make_async_copy(
                rows_v.at[b],
                out_hbm.at[base + r],
                ssem.at[b],
            )

        def scale(b):
            @pl.loop(0, xt, step=ROW_UNROLL)
            def _rows(r):
                for dr in range(ROW_UNROLL):
                    for j in range(D_EMBED // LANES):
                        sl = pl.ds(j * LANES, LANES)
                        rows_v[b, r + dr, sl] = rows_v[b, r + dr, sl] * SCALE

        gather(0, 0).start()

        @pl.loop(0, rows_per_worker // NBUF)
        def _group(g):
            r0 = g * NBUF
            for b in range(NBUF):
                r = r0 + b
                nb = (b + 1) % NBUF

                @pl.when(r + 1 < rows_per_worker)
                def _start_next():
                    @pl.when(r >= NBUF - 1)
                    def _drain_nb():
                        store(0, nb).wait()

                    gather(r + 1, nb).start()

                gather(r, b).wait()
                scale(b)
                store(r, b).start()

        for b in range(NBUF):
            store(0, b).wait()

    return sc_gather


def kernel(x, table):
    b, t = x.shape
    return _build_sc_gather(b, t)(x.astype(jnp.int32), table)
